# baseline (speedup 1.0000x reference)
import jax
import jax.numpy as jnp
from jax import lax
from jax.experimental import pallas as pl
from jax.experimental.pallas import tpu as pltpu
from jax.experimental.pallas import tpu_sc as plsc

B, C, T, E = 16, 64, 1024, 64
QL, NCLS = 1024, 100
NB = 8
TT = T // 4
TS = 64
L = 16


def _body(x_hbm, chs_hbm, cond_hbm, qw_hbm, cw_hbm, out_hbm,
          qt, cht, cwt, cmask, obuf, xbuf, cibuf, xsem, osem0, osem1):
    wid = lax.axis_index("s") * 2 + lax.axis_index("c")
    b = wid // 4
    h = wid % 4
    t0 = h * TT
    osems = (osem0, osem1)

    pltpu.sync_copy(qw_hbm, qt)
    pltpu.sync_copy(chs_hbm, cht)
    pltpu.sync_copy(cw_hbm, cwt)
    pltpu.sync_copy(cond_hbm.at[b, pl.ds(t0, TT)], cibuf)

    @plsc.parallel_loop(0, TT, step=L)
    def cond_group(tl):
        civ = cibuf[pl.ds(tl, L)]
        for k in range(L):
            ci = civ[k]
            m = jnp.where(ci > 0, 1.0, 0.0).astype(jnp.float32)
            cr = ci >> 1
            cp = (ci & 1) * E
            for j in range(E // L):
                cmask[(tl + k) // 2, pl.ds((k % 2) * E + j * L, L)] = (
                    cwt[cr, pl.ds(cp + j * L, L)] * m)

    pltpu.async_copy(x_hbm.at[b, 0, pl.ds(t0, TT)], xbuf.at[pl.ds(0, TT)],
                     xsem)

    def owait(p):
        pltpu.make_async_copy(
            obuf.at[p], out_hbm.at[b * C, pl.ds(t0, TS), :], osems[p]).wait()

    def chan_body(cc, _):
        xoff = (cc % 2) * TT

        @pl.when(cc + 1 < C)
        def _():
            pltpu.async_copy(
                x_hbm.at[b, cc + 1, pl.ds(t0, TT)],
                xbuf.at[pl.ds(((cc + 1) % 2) * TT, TT)], xsem)

        pltpu.make_async_copy(
            x_hbm.at[b, cc, pl.ds(t0, TT)],
            xbuf.at[pl.ds(xoff, TT)], xsem).wait()

        chr_ = cc >> 1
        chp = (cc & 1) * E
        ch = [cht[chr_, pl.ds(chp + j * L, L)] for j in range(E // L)]

        for s in range(TT // TS):
            p = s % 2
            if s < 2:
                @pl.when(cc > 0)
                def _():
                    owait(p)
            else:
                owait(p)

            @plsc.parallel_loop(0, TS, step=L)
            def row_group(tl):
                t = s * TS + tl
                xv = xbuf[pl.ds(xoff + t, L)]
                for kb in range(0, L, 4):
                    qs, cms = [], []
                    for k in range(kb, kb + 4):
                        ix = xv[k]
                        qr = ix >> 1
                        qp = (ix & 1) * E
                        qs.append([qt[qr, pl.ds(qp + j * L, L)]
                                   for j in range(E // L)])
                        cms.append([cmask[(t + k) // 2,
                                          pl.ds((k % 2) * E + j * L, L)]
                                    for j in range(E // L)])
                    outs = [[qs[i][j] + cms[i][j] + ch[j]
                             for j in range(E // L)]
                            for i in range(4)]
                    for i, k in enumerate(range(kb, kb + 4)):
                        for j in range(E // L):
                            obuf[p, tl + k, pl.ds(j * L, L)] = outs[i][j]

            pltpu.async_copy(
                obuf.at[p],
                out_hbm.at[b * C + cc, pl.ds(t0 + s * TS, TS), :], osems[p])
        return 0

    lax.fori_loop(0, C, chan_body, 0)
    owait(0)
    owait(1)


@jax.jit
def _run(x, ch_sel, cond, quant_W, cond_W):
    mesh = plsc.VectorSubcoreMesh(core_axis_name="c", subcore_axis_name="s")
    f = pl.kernel(
        _body,
        out_type=jax.ShapeDtypeStruct((NB * C, T, E), jnp.float32),
        mesh=mesh,
        compiler_params=pltpu.CompilerParams(use_tc_tiling_on_sc=True),
        scratch_types=[
            pltpu.VMEM((QL // 2, 2 * E), jnp.float32),
            pltpu.VMEM((C // 2, 2 * E), jnp.float32),
            pltpu.VMEM((NCLS // 2, 2 * E), jnp.float32),
            pltpu.VMEM((TT // 2, 2 * E), jnp.float32),
            pltpu.VMEM((2, TS, E), jnp.float32),
            pltpu.VMEM((2 * TT,), jnp.int32),
            pltpu.VMEM((TT,), jnp.int32),
            pltpu.SemaphoreType.DMA,
            pltpu.SemaphoreType.DMA,
            pltpu.SemaphoreType.DMA,
        ],
    )
    return f(x, ch_sel, cond, quant_W, cond_W)


def kernel(x, ids, cond, quant_W, channel_W, cond_W):
    x = x.astype(jnp.int32)
    cond = cond.astype(jnp.int32).reshape(B, T)
    ch_sel = jnp.take(channel_W, ids.astype(jnp.int32), axis=0)
    ch_sel = ch_sel.reshape(C // 2, 2 * E)
    qw = quant_W.reshape(QL // 2, 2 * E)
    cw = cond_W.reshape(NCLS // 2, 2 * E)
    o1 = _run(x[:NB], ch_sel, cond[:NB], qw, cw)
    o2 = _run(x[NB:], ch_sel, cond[NB:], qw, cw)
    return jnp.concatenate([o1, o2], axis=0)


# speedup vs baseline: 1.1129x; 1.1129x over previous
import jax
import jax.numpy as jnp
from jax import lax
from jax.experimental import pallas as pl
from jax.experimental.pallas import tpu as pltpu
from jax.experimental.pallas import tpu_sc as plsc

B, C, T, E = 16, 64, 1024, 64
QL, NCLS = 1024, 100
TT = T // 2
TS = 64
L = 16


def _body(x_hbm, chs_hbm, cond_hbm, qw_hbm, cw_hbm, out_hbm,
          qt, cht, cwt, cmask, obuf, xbuf, cibuf, xsem, osem0, osem1):
    wid = lax.axis_index("s") * 2 + lax.axis_index("c")
    b = wid // 2
    h = wid % 2
    t0 = h * TT
    osems = (osem0, osem1)

    pltpu.sync_copy(qw_hbm, qt)
    pltpu.sync_copy(chs_hbm, cht)
    pltpu.sync_copy(cw_hbm, cwt)
    pltpu.sync_copy(cond_hbm.at[b, pl.ds(t0, TT)], cibuf)

    @plsc.parallel_loop(0, TT, step=L)
    def cond_group(tl):
        civ = cibuf[pl.ds(tl, L)]
        for k in range(L):
            ci = civ[k]
            m = jnp.where(ci > 0, 1.0, 0.0).astype(jnp.float32)
            cr = ci >> 1
            cp = (ci & 1) * E
            for j in range(E // L):
                cmask[(tl + k) // 2, pl.ds((k % 2) * E + j * L, L)] = (
                    cwt[cr, pl.ds(cp + j * L, L)] * m)

    pltpu.async_copy(x_hbm.at[b, 0, pl.ds(t0, TT)], xbuf.at[pl.ds(0, TT)],
                     xsem)

    def owait(p):
        pltpu.make_async_copy(
            obuf.at[p], out_hbm.at[b * C, pl.ds(t0, TS), :], osems[p]).wait()

    def chan_body(cc, _):
        xoff = (cc % 2) * TT

        @pl.when(cc + 1 < C)
        def _():
            pltpu.async_copy(
                x_hbm.at[b, cc + 1, pl.ds(t0, TT)],
                xbuf.at[pl.ds(((cc + 1) % 2) * TT, TT)], xsem)

        pltpu.make_async_copy(
            x_hbm.at[b, cc, pl.ds(t0, TT)],
            xbuf.at[pl.ds(xoff, TT)], xsem).wait()

        chr_ = cc >> 1
        chp = (cc & 1) * E
        ch = [cht[chr_, pl.ds(chp + j * L, L)] for j in range(E // L)]

        for s in range(TT // TS):
            p = s % 2
            if s < 2:
                @pl.when(cc > 0)
                def _():
                    owait(p)
            else:
                owait(p)

            @plsc.parallel_loop(0, TS, step=L)
            def row_group(tl):
                t = s * TS + tl
                xv = xbuf[pl.ds(xoff + t, L)]
                for kb in range(0, L, 4):
                    qs, cms = [], []
                    for k in range(kb, kb + 4):
                        ix = xv[k]
                        qr = ix >> 1
                        qp = (ix & 1) * E
                        qs.append([qt[qr, pl.ds(qp + j * L, L)]
                                   for j in range(E // L)])
                        cms.append([cmask[(t + k) // 2,
                                          pl.ds((k % 2) * E + j * L, L)]
                                    for j in range(E // L)])
                    outs = [[qs[i][j] + cms[i][j] + ch[j]
                             for j in range(E // L)]
                            for i in range(4)]
                    for i, k in enumerate(range(kb, kb + 4)):
                        for j in range(E // L):
                            obuf[p, tl + k, pl.ds(j * L, L)] = outs[i][j]

            pltpu.async_copy(
                obuf.at[p],
                out_hbm.at[b * C + cc, pl.ds(t0 + s * TS, TS), :], osems[p])
        return 0

    lax.fori_loop(0, C, chan_body, 0)
    owait(0)
    owait(1)


@jax.jit
def _run(x, ch_sel, cond, quant_W, cond_W):
    mesh = plsc.VectorSubcoreMesh(core_axis_name="c", subcore_axis_name="s")
    f = pl.kernel(
        _body,
        out_type=jax.ShapeDtypeStruct((B * C, T, E), jnp.float32),
        mesh=mesh,
        compiler_params=pltpu.CompilerParams(use_tc_tiling_on_sc=True),
        scratch_types=[
            pltpu.VMEM((QL // 2, 2 * E), jnp.float32),
            pltpu.VMEM((C // 2, 2 * E), jnp.float32),
            pltpu.VMEM((NCLS // 2, 2 * E), jnp.float32),
            pltpu.VMEM((TT // 2, 2 * E), jnp.float32),
            pltpu.VMEM((2, TS, E), jnp.float32),
            pltpu.VMEM((2 * TT,), jnp.int32),
            pltpu.VMEM((TT,), jnp.int32),
            pltpu.SemaphoreType.DMA,
            pltpu.SemaphoreType.DMA,
            pltpu.SemaphoreType.DMA,
        ],
    )
    return f(x, ch_sel, cond, quant_W, cond_W)


def kernel(x, ids, cond, quant_W, channel_W, cond_W):
    x = x.astype(jnp.int32)
    cond = cond.astype(jnp.int32).reshape(B, T)
    ch_sel = jnp.take(channel_W, ids.astype(jnp.int32), axis=0)
    ch_sel = ch_sel.reshape(C // 2, 2 * E)
    qw = quant_W.reshape(QL // 2, 2 * E)
    cw = cond_W.reshape(NCLS // 2, 2 * E)
    return _run(x, ch_sel, cond, qw, cw)
